# baseline (speedup 1.0000x reference)
import jax
import jax.numpy as jnp
from jax.experimental import pallas as pl


def _body(tab_ref, w1_ref, b1_ref, w2_ref, b2_ref, out_ref):
    prompt = tab_ref[:, :]
    h = jnp.tanh(
        jnp.dot(prompt, w1_ref[:, :], preferred_element_type=jnp.float32)
        + b1_ref[:].reshape(1, -1)
    )
    out_ref[:, :] = (
        jnp.dot(h, w2_ref[:, :], preferred_element_type=jnp.float32)
        + b2_ref[:].reshape(1, -1)
    )


def kernel(tokens, batch_size, pre_prompt, embd_table, W1, b1, W2, b2):
    B = tokens.shape[0]
    P = pre_prompt.shape[0]
    D, H = W1.shape
    res = pl.pallas_call(
        _body,
        out_shape=jax.ShapeDtypeStruct((P, D), jnp.float32),
    )(embd_table, W1, b1, W2, b2)
    return jnp.concatenate([res[None]] * B, axis=0)
